# baseline (speedup 1.0000x reference)
import functools

import jax
import jax.numpy as jnp
from jax import lax
from jax.experimental import pallas as pl
from jax.experimental.pallas import tpu as pltpu
from jax.experimental.pallas import tpu_sc as plsc

N_CAND = 8
B, S, D = 2, 2048, 1024
SLAB_ROWS = B * S
TOTAL_ROWS = N_CAND * SLAB_ROWS

NW = 32
ROWS_PER_W = SLAB_ROWS // NW
CHUNK_ROWS = 8
N_CHUNKS = ROWS_PER_W // CHUNK_ROWS
NSET = 7
LANES = 16
COL_GROUPS = D // LANES
ROW_GROUPS = CHUNK_ROWS * COL_GROUPS


def _avg_pair_impl(in_ref, dsel_ref, out_ref, dvec, *rest):
    cid = lax.axis_index("c")
    sid = lax.axis_index("s")
    wid = sid * 2 + cid

    pltpu.sync_copy(dsel_ref, dvec)
    d = dvec[...][0]
    arow = d * (2 * SLAB_ROWS) + wid * ROWS_PER_W
    brow = arow + SLAB_ROWS
    orow = wid * ROWS_PER_W

    abufs = rest[0:2 * NSET:2]
    bbufs = rest[1:2 * NSET:2]
    sas = rest[2 * NSET:3 * NSET]
    sbs = rest[3 * NSET:4 * NSET]
    sos = rest[4 * NSET:5 * NSET]
    sets = tuple(zip(abufs, bbufs, sas, sbs, sos))

    def start_in(g):
        a, b, sa, sb, _ = sets[g % NSET]
        off = g * CHUNK_ROWS
        da = pltpu.async_copy(in_ref.at[pl.ds(arow + off, CHUNK_ROWS)], a, sa)
        db = pltpu.async_copy(in_ref.at[pl.ds(brow + off, CHUNK_ROWS)], b, sb)
        return da, db

    half = jnp.float32(0.5)
    PREF = NSET - 1
    pend = [None] * N_CHUNKS
    out_dmas = [None] * N_CHUNKS
    for k in range(min(PREF, N_CHUNKS)):
        pend[k] = start_in(k)
    for g in range(N_CHUNKS):
        a, b, _, _, so = sets[g % NSET]
        nx = g + PREF
        if nx < N_CHUNKS:
            if g >= 1 and out_dmas[g - 1] is not None:
                out_dmas[g - 1].wait()
                out_dmas[g - 1] = None
            pend[nx] = start_in(nx)
        pend[g][0].wait()
        pend[g][1].wait()

        @plsc.parallel_loop(0, ROW_GROUPS, step=1, unroll=4)
        def _(i):
            r = i >> 6
            c = (i & (COL_GROUPS - 1)) * LANES
            a[r, pl.ds(c, LANES)] = (
                a[r, pl.ds(c, LANES)] + b[r, pl.ds(c, LANES)]) * half

        out_dmas[g] = pltpu.async_copy(
            a, out_ref.at[pl.ds(orow + g * CHUNK_ROWS, CHUNK_ROWS)], so)

    for od in out_dmas:
        if od is not None:
            od.wait()


@functools.lru_cache(maxsize=1)
def _build_avg_pair():
    mesh = plsc.VectorSubcoreMesh(core_axis_name="c", subcore_axis_name="s")
    return pl.kernel(
        _avg_pair_impl,
        out_type=jax.ShapeDtypeStruct((SLAB_ROWS, D), jnp.float32),
        mesh=mesh,
        compiler_params=pltpu.CompilerParams(
            use_tc_tiling_on_sc=True,
            skip_device_barrier=True,
            disable_bounds_checks=True,
            disable_semaphore_checks=True,
        ),
        scratch_types=(
            [pltpu.VMEM((LANES,), jnp.int32)]
            + [pltpu.VMEM((CHUNK_ROWS, D), jnp.float32)] * (2 * NSET)
            + [pltpu.SemaphoreType.DMA] * (3 * NSET)
        ),
    )


def kernel(inputs, domain_idx):
    rows = inputs.reshape(TOTAL_ROWS, D)
    dsel = jnp.full((LANES,), jnp.asarray(domain_idx, jnp.int32), jnp.int32)
    out = _build_avg_pair()(rows, dsel)
    return out.reshape(B, S, D)


# speedup vs baseline: 1.0004x; 1.0004x over previous
import functools

import jax
import jax.numpy as jnp
from jax import lax
from jax.experimental import pallas as pl
from jax.experimental.pallas import tpu as pltpu
from jax.experimental.pallas import tpu_sc as plsc

N_CAND = 8
B, S, D = 2, 2048, 1024
SLAB_ROWS = B * S
TOTAL_ROWS = N_CAND * SLAB_ROWS

NW = 32
ROWS_PER_W = SLAB_ROWS // NW
CHUNK_ROWS = 8
N_CHUNKS = ROWS_PER_W // CHUNK_ROWS
NSET = 6
LANES = 16
COL_GROUPS = D // LANES
ROW_GROUPS = CHUNK_ROWS * COL_GROUPS


def _avg_pair_impl(in_ref, dsel_ref, out_ref, dvec, *rest):
    cid = lax.axis_index("c")
    sid = lax.axis_index("s")
    wid = sid * 2 + cid

    pltpu.sync_copy(dsel_ref, dvec)
    d = dvec[...][0]
    arow = d * (2 * SLAB_ROWS) + wid * ROWS_PER_W
    brow = arow + SLAB_ROWS
    orow = wid * ROWS_PER_W

    abufs = rest[0:2 * NSET:2]
    bbufs = rest[1:2 * NSET:2]
    sas = rest[2 * NSET:3 * NSET]
    sbs = rest[3 * NSET:4 * NSET]
    sos = rest[4 * NSET:5 * NSET]
    sets = tuple(zip(abufs, bbufs, sas, sbs, sos))

    def start_in(g):
        a, b, sa, sb, _ = sets[g % NSET]
        off = g * CHUNK_ROWS
        da = pltpu.async_copy(in_ref.at[pl.ds(arow + off, CHUNK_ROWS)], a, sa)
        db = pltpu.async_copy(in_ref.at[pl.ds(brow + off, CHUNK_ROWS)], b, sb)
        return da, db

    half = jnp.float32(0.5)
    PREF = NSET - 2
    pend = [None] * N_CHUNKS
    out_dmas = [None] * N_CHUNKS
    for k in range(min(PREF, N_CHUNKS)):
        pend[k] = start_in(k)
    for g in range(N_CHUNKS):
        a, b, _, _, so = sets[g % NSET]
        nx = g + PREF
        if nx < N_CHUNKS:
            prev = nx - NSET
            if prev >= 0 and out_dmas[prev] is not None:
                out_dmas[prev].wait()
                out_dmas[prev] = None
            pend[nx] = start_in(nx)
        pend[g][0].wait()
        pend[g][1].wait()

        @plsc.parallel_loop(0, ROW_GROUPS, step=1, unroll=4)
        def _(i):
            r = i >> 6
            c = (i & (COL_GROUPS - 1)) * LANES
            a[r, pl.ds(c, LANES)] = (
                a[r, pl.ds(c, LANES)] + b[r, pl.ds(c, LANES)]) * half

        out_dmas[g] = pltpu.async_copy(
            a, out_ref.at[pl.ds(orow + g * CHUNK_ROWS, CHUNK_ROWS)], so)

    for od in out_dmas:
        if od is not None:
            od.wait()


@functools.lru_cache(maxsize=1)
def _build_avg_pair():
    mesh = plsc.VectorSubcoreMesh(core_axis_name="c", subcore_axis_name="s")
    return pl.kernel(
        _avg_pair_impl,
        out_type=jax.ShapeDtypeStruct((SLAB_ROWS, D), jnp.float32),
        mesh=mesh,
        compiler_params=pltpu.CompilerParams(
            use_tc_tiling_on_sc=True,
            skip_device_barrier=True,
            disable_bounds_checks=True,
            disable_semaphore_checks=True,
        ),
        scratch_types=(
            [pltpu.VMEM((LANES,), jnp.int32)]
            + [pltpu.VMEM((CHUNK_ROWS, D), jnp.float32)] * (2 * NSET)
            + [pltpu.SemaphoreType.DMA] * (3 * NSET)
        ),
    )


def kernel(inputs, domain_idx):
    rows = inputs.reshape(TOTAL_ROWS, D)
    dsel = jnp.full((LANES,), jnp.asarray(domain_idx, jnp.int32), jnp.int32)
    out = _build_avg_pair()(rows, dsel)
    return out.reshape(B, S, D)


# speedup vs baseline: 1.0268x; 1.0264x over previous
import functools

import jax
import jax.numpy as jnp
from jax import lax
from jax.experimental import pallas as pl
from jax.experimental.pallas import tpu as pltpu
from jax.experimental.pallas import tpu_sc as plsc

N_CAND = 8
B, S, D = 2, 2048, 1024
SLAB_ROWS = B * S
TOTAL_ROWS = N_CAND * SLAB_ROWS

NW = 32
ROWS_PER_W = SLAB_ROWS // NW
CHUNK_ROWS = 8
N_CHUNKS = ROWS_PER_W // CHUNK_ROWS
NSET = 6
LANES = 16
COL_GROUPS = D // LANES
ROW_GROUPS = CHUNK_ROWS * COL_GROUPS


def _avg_pair_impl(in_ref, dsel_ref, out_ref, dvec, *rest):
    cid = lax.axis_index("c")
    sid = lax.axis_index("s")
    wid = sid * 2 + cid

    pltpu.sync_copy(dsel_ref, dvec)
    d = dvec[...][0]
    arow = d * (2 * SLAB_ROWS) + wid * ROWS_PER_W
    brow = arow + SLAB_ROWS
    orow = wid * ROWS_PER_W

    abufs = rest[0:2 * NSET:2]
    bbufs = rest[1:2 * NSET:2]
    sas = rest[2 * NSET:3 * NSET]
    sbs = rest[3 * NSET:4 * NSET]
    sos = rest[4 * NSET:5 * NSET]
    sets = tuple(zip(abufs, bbufs, sas, sbs, sos))

    def start_in(g):
        a, b, sa, sb, _ = sets[g % NSET]
        off = g * CHUNK_ROWS
        da = pltpu.async_copy(in_ref.at[pl.ds(arow + off, CHUNK_ROWS)], a, sa)
        db = pltpu.async_copy(in_ref.at[pl.ds(brow + off, CHUNK_ROWS)], b, sb)
        return da, db

    half = jnp.float32(0.5)
    PREF = NSET - 1
    pend = [None] * N_CHUNKS
    out_dmas = [None] * N_CHUNKS
    for k in range(min(PREF, N_CHUNKS)):
        pend[k] = start_in(k)
    for g in range(N_CHUNKS):
        a, b, _, _, so = sets[g % NSET]
        nx = g + PREF
        if nx < N_CHUNKS:
            prev = nx - NSET
            if prev >= 0 and out_dmas[prev] is not None:
                out_dmas[prev].wait()
                out_dmas[prev] = None
            pend[nx] = start_in(nx)
        pend[g][0].wait()
        pend[g][1].wait()

        @plsc.parallel_loop(0, ROW_GROUPS, step=1, unroll=4)
        def _(i):
            r = i >> 6
            c = (i & (COL_GROUPS - 1)) * LANES
            a[r, pl.ds(c, LANES)] = (
                a[r, pl.ds(c, LANES)] + b[r, pl.ds(c, LANES)]) * half

        out_dmas[g] = pltpu.async_copy(
            a, out_ref.at[pl.ds(orow + g * CHUNK_ROWS, CHUNK_ROWS)], so)

    for od in out_dmas:
        if od is not None:
            od.wait()


@functools.lru_cache(maxsize=1)
def _build_avg_pair():
    mesh = plsc.VectorSubcoreMesh(core_axis_name="c", subcore_axis_name="s")
    return pl.kernel(
        _avg_pair_impl,
        out_type=jax.ShapeDtypeStruct((SLAB_ROWS, D), jnp.float32),
        mesh=mesh,
        compiler_params=pltpu.CompilerParams(
            use_tc_tiling_on_sc=True,
            skip_device_barrier=True,
            disable_bounds_checks=True,
            disable_semaphore_checks=True,
        ),
        scratch_types=(
            [pltpu.VMEM((LANES,), jnp.int32)]
            + [pltpu.VMEM((CHUNK_ROWS, D), jnp.float32)] * (2 * NSET)
            + [pltpu.SemaphoreType.DMA] * (3 * NSET)
        ),
    )


def kernel(inputs, domain_idx):
    rows = inputs.reshape(TOTAL_ROWS, D)
    dsel = jnp.full((LANES,), jnp.asarray(domain_idx, jnp.int32), jnp.int32)
    out = _build_avg_pair()(rows, dsel)
    return out.reshape(B, S, D)
